# Initial kernel scaffold; baseline (speedup 1.0000x reference)
import functools

import jax
import jax.numpy as jnp
from jax import lax
from jax.experimental import pallas as pl
from jax.experimental.pallas import tpu as pltpu
from jax.experimental.pallas import tpu_sc as plsc

EPS = 1e-15
MAX_TANH = 15.0

NC = 2
NS = 16
NW = NC * NS


def _artanh(x):
    x = jnp.clip(x, -1 + 1e-7, 1 - 1e-7)
    return 0.5 * (jnp.log1p(x) - jnp.log1p(-x))


def _safe_norm(x):
    return jnp.sqrt(jnp.clip(jnp.sum(x * x, axis=-1, keepdims=True), EPS, None))


def _proj(x):
    norm = _safe_norm(x)
    maxnorm = 1.0 - 1e-3
    return jnp.where(norm > maxnorm, x / norm * maxnorm, x)


def _expmap0(u):
    u_norm = _safe_norm(u)
    return jnp.tanh(jnp.clip(u_norm, -MAX_TANH, MAX_TANH)) * u / u_norm


def _logmap0(x):
    x_norm = _safe_norm(x)
    return _artanh(x_norm) * x / x_norm


def _head_body(x_ref, wt_ref, b_ref, o_ref):
    x = x_ref[...]
    x_hyp = _proj(x)
    x_norm = _safe_norm(x_hyp)
    mx = jnp.dot(x_hyp, wt_ref[...], preferred_element_type=jnp.float32)
    mx_norm = _safe_norm(mx)
    mv = jnp.tanh(jnp.clip(mx_norm / x_norm * _artanh(x_norm),
                           -MAX_TANH, MAX_TANH)) * mx / mx_norm
    mv = _proj(mv)
    bias = _proj(_expmap0(b_ref[...]))
    x2 = jnp.sum(mv * mv, axis=-1, keepdims=True)
    y2 = jnp.sum(bias * bias, axis=-1, keepdims=True)
    xy = jnp.sum(mv * bias, axis=-1, keepdims=True)
    num = (1 + 2 * xy + y2) * mv + (1 - x2) * bias
    den = 1 + 2 * xy + x2 * y2
    h = _proj(num / jnp.clip(den, EPS, None))
    o_ref[...] = _logmap0(h)


def _tail_body(a_ref, b_ref, o_ref):
    agg = a_ref[...] + b_ref[...]
    h_agg = _proj(_expmap0(agg))
    out = _proj(_expmap0(jax.nn.relu(_logmap0(h_agg))))
    o_ref[...] = _logmap0(out)


def _sc_aggregate(x_tan, src, dst, zeros, *, n, d, acc_rows, ept, k):
    chunks = ept // k
    rpt_zero = acc_rows // NS
    rpt_out = n // NS
    mesh = plsc.VectorSubcoreMesh(core_axis_name="c", subcore_axis_name="s")

    @functools.partial(
        pl.kernel,
        mesh=mesh,
        out_type=jax.ShapeDtypeStruct((NC, n, d), jnp.float32),
        scratch_types=[
            pltpu.VMEM_SHARED((acc_rows, d), jnp.float32),
            pltpu.VMEM((k,), jnp.int32),
            pltpu.VMEM((k,), jnp.int32),
            pltpu.VMEM((k, d), jnp.float32),
            pltpu.SemaphoreType.DMA,
        ],
    )
    def agg_kernel(xtan_hbm, src_hbm, dst_hbm, zeros_hbm, out_hbm,
                   acc, src_v, dst_v, rows_v, sem):
        c = lax.axis_index("c")
        s = lax.axis_index("s")
        wid = s * NC + c
        z0 = pl.multiple_of(s * rpt_zero, 8)
        pltpu.sync_copy(zeros_hbm.at[pl.ds(z0, rpt_zero)],
                        acc.at[pl.ds(z0, rpt_zero)])
        plsc.subcore_barrier()

        base0 = wid * ept

        def body(g, carry):
            base = pl.multiple_of(base0 + g * k, 8)
            pltpu.sync_copy(src_hbm.at[pl.ds(base, k)], src_v)
            pltpu.sync_copy(dst_hbm.at[pl.ds(base, k)], dst_v)
            pltpu.async_copy(xtan_hbm.at[src_v], rows_v, sem).wait()
            pltpu.sync_copy(rows_v, acc.at[dst_v], add=True)
            return carry

        lax.fori_loop(0, chunks, body, 0)
        plsc.subcore_barrier()
        o0 = pl.multiple_of(s * rpt_out, 8)
        pltpu.sync_copy(acc.at[pl.ds(o0, rpt_out)],
                        out_hbm.at[c, pl.ds(o0, rpt_out)])

    return agg_kernel(x_tan, src, dst, zeros)


def kernel(x, adj, W, b):
    n, d = x.shape
    e = adj.shape[1]
    assert n % NS == 0 and d % 128 == 0

    rb = 500
    grid_h = n // rb
    x_tan = pl.pallas_call(
        _head_body,
        grid=(grid_h,),
        in_specs=[
            pl.BlockSpec((rb, d), lambda i: (i, 0)),
            pl.BlockSpec((d, d), lambda i: (0, 0)),
            pl.BlockSpec((1, d), lambda i: (0, 0)),
        ],
        out_specs=pl.BlockSpec((rb, d), lambda i: (i, 0)),
        out_shape=jax.ShapeDtypeStruct((n, d), jnp.float32),
    )(x, W.T, b.reshape(1, d))

    k = 128
    e_pad = ((e + NW * k - 1) // (NW * k)) * (NW * k)
    src = adj[0].astype(jnp.int32)
    dst = adj[1].astype(jnp.int32)
    if e_pad != e:
        src = jnp.pad(src, (0, e_pad - e))
        dst = jnp.pad(dst, (0, e_pad - e), constant_values=n)
    acc_rows = ((n + 1 + NS - 1) // NS) * NS
    zeros = jnp.zeros((acc_rows, d), jnp.float32)
    partials = _sc_aggregate(x_tan, src, dst, zeros,
                             n=n, d=d, acc_rows=acc_rows,
                             ept=e_pad // NW, k=k)

    out = pl.pallas_call(
        _tail_body,
        grid=(grid_h,),
        in_specs=[
            pl.BlockSpec((rb, d), lambda i: (i, 0)),
            pl.BlockSpec((rb, d), lambda i: (i, 0)),
        ],
        out_specs=pl.BlockSpec((rb, d), lambda i: (i, 0)),
        out_shape=jax.ShapeDtypeStruct((n, d), jnp.float32),
    )(partials[0], partials[1])
    return out


# speedup vs baseline: 4.0224x; 4.0224x over previous
import functools

import jax
import jax.numpy as jnp
from jax import lax
from jax.experimental import pallas as pl
from jax.experimental.pallas import tpu as pltpu
from jax.experimental.pallas import tpu_sc as plsc

EPS = 1e-15
MAX_TANH = 15.0

NC = 2
NS = 16
NW = NC * NS


def _artanh(x):
    x = jnp.clip(x, -1 + 1e-7, 1 - 1e-7)
    return 0.5 * (jnp.log1p(x) - jnp.log1p(-x))


def _safe_norm(x):
    return jnp.sqrt(jnp.clip(jnp.sum(x * x, axis=-1, keepdims=True), EPS, None))


def _proj(x):
    norm = _safe_norm(x)
    maxnorm = 1.0 - 1e-3
    return jnp.where(norm > maxnorm, x / norm * maxnorm, x)


def _expmap0(u):
    u_norm = _safe_norm(u)
    return jnp.tanh(jnp.clip(u_norm, -MAX_TANH, MAX_TANH)) * u / u_norm


def _logmap0(x):
    x_norm = _safe_norm(x)
    return _artanh(x_norm) * x / x_norm


def _head_body(x_ref, wt_ref, b_ref, o_ref):
    x = x_ref[...]
    x_hyp = _proj(x)
    x_norm = _safe_norm(x_hyp)
    mx = jnp.dot(x_hyp, wt_ref[...], preferred_element_type=jnp.float32)
    mx_norm = _safe_norm(mx)
    mv = jnp.tanh(jnp.clip(mx_norm / x_norm * _artanh(x_norm),
                           -MAX_TANH, MAX_TANH)) * mx / mx_norm
    mv = _proj(mv)
    bias = _proj(_expmap0(b_ref[...]))
    x2 = jnp.sum(mv * mv, axis=-1, keepdims=True)
    y2 = jnp.sum(bias * bias, axis=-1, keepdims=True)
    xy = jnp.sum(mv * bias, axis=-1, keepdims=True)
    num = (1 + 2 * xy + y2) * mv + (1 - x2) * bias
    den = 1 + 2 * xy + x2 * y2
    h = _proj(num / jnp.clip(den, EPS, None))
    o_ref[...] = _logmap0(h)


def _tail_body(a_ref, b_ref, o_ref):
    agg = a_ref[...] + b_ref[...]
    h_agg = _proj(_expmap0(agg))
    out = _proj(_expmap0(jax.nn.relu(_logmap0(h_agg))))
    o_ref[...] = _logmap0(out)


def _sc_aggregate(x_tan, src, dst, zeros, *, n, d, acc_rows, ept, k):
    chunks = ept // k
    rpt = acc_rows // NS
    mesh = plsc.VectorSubcoreMesh(core_axis_name="c", subcore_axis_name="s")

    @functools.partial(
        pl.kernel,
        mesh=mesh,
        out_type=jax.ShapeDtypeStruct((NC, acc_rows, d), jnp.float32),
        scratch_types=[
            pltpu.VMEM_SHARED((acc_rows, d), jnp.float32),
            pltpu.VMEM((k,), jnp.int32),
            pltpu.VMEM((k,), jnp.int32),
            pltpu.VMEM((k, d), jnp.float32),
            pltpu.SemaphoreType.DMA,
        ],
    )
    def agg_kernel(xtan_hbm, src_hbm, dst_hbm, zeros_hbm, out_hbm,
                   acc, src_v, dst_v, rows_v, sem):
        c = lax.axis_index("c")
        s = lax.axis_index("s")
        wid = s * NC + c
        z0 = pl.multiple_of(s * rpt, 8)
        pltpu.sync_copy(zeros_hbm.at[pl.ds(z0, rpt)],
                        acc.at[pl.ds(z0, rpt)])
        plsc.subcore_barrier()

        base0 = wid * ept

        def body(g, carry):
            base = pl.multiple_of(base0 + g * k, 8)
            pltpu.sync_copy(src_hbm.at[pl.ds(base, k)], src_v)
            pltpu.sync_copy(dst_hbm.at[pl.ds(base, k)], dst_v)
            pltpu.async_copy(xtan_hbm.at[src_v], rows_v, sem).wait()
            pltpu.sync_copy(rows_v, acc.at[dst_v], add=True)
            return carry

        lax.fori_loop(0, chunks, body, 0)
        plsc.subcore_barrier()
        pltpu.sync_copy(acc.at[pl.ds(z0, rpt)],
                        out_hbm.at[c, pl.ds(z0, rpt)])

    return agg_kernel(x_tan, src, dst, zeros)


def kernel(x, adj, W, b):
    n, d = x.shape
    e = adj.shape[1]
    assert n % NS == 0 and d % 128 == 0

    rb = 400
    grid_h = n // rb
    x_tan = pl.pallas_call(
        _head_body,
        grid=(grid_h,),
        in_specs=[
            pl.BlockSpec((rb, d), lambda i: (i, 0)),
            pl.BlockSpec((d, d), lambda i: (0, 0)),
            pl.BlockSpec((1, d), lambda i: (0, 0)),
        ],
        out_specs=pl.BlockSpec((rb, d), lambda i: (i, 0)),
        out_shape=jax.ShapeDtypeStruct((n, d), jnp.float32),
    )(x, W.T, b.reshape(1, d))

    k = 128
    e_pad = ((e + NW * k - 1) // (NW * k)) * (NW * k)
    src = adj[0].astype(jnp.int32)
    dst = adj[1].astype(jnp.int32)
    if e_pad != e:
        src = jnp.pad(src, (0, e_pad - e))
        dst = jnp.pad(dst, (0, e_pad - e), constant_values=n)
    acc_rows = ((n + 1 + NS * 8 - 1) // (NS * 8)) * (NS * 8)
    zeros = jnp.zeros((acc_rows, d), jnp.float32)
    partials = _sc_aggregate(x_tan, src, dst, zeros,
                             n=n, d=d, acc_rows=acc_rows,
                             ept=e_pad // NW, k=k)

    out = pl.pallas_call(
        _tail_body,
        grid=(grid_h,),
        in_specs=[
            pl.BlockSpec((rb, d), lambda i: (i, 0)),
            pl.BlockSpec((rb, d), lambda i: (i, 0)),
        ],
        out_specs=pl.BlockSpec((rb, d), lambda i: (i, 0)),
        out_shape=jax.ShapeDtypeStruct((n, d), jnp.float32),
    )(partials[0], partials[1])
    return out
